# baseline (speedup 1.0000x reference)
import jax
import jax.numpy as jnp
from jax import lax
from jax.experimental import pallas as pl
from jax.experimental.pallas import tpu as pltpu

_T, _B, _N, _FEAT, _R, _EMB = 2, 8, 300, 32, 3, 16


def _dot_t(a, b):
    return lax.dot_general(a, b, (((0,), (0,)), ((), ())),
                           preferred_element_type=jnp.float32)


def _mm(a, b):
    return jnp.dot(a, b, preferred_element_type=jnp.float32)


def _gnn_kernel(xt_ref, adj_ref, embw_ref, embb_ref,
                w1_ref, r1_ref, b1_ref, w2_ref, r2_ref, b2_ref, out_ref,
                planes_ref, h_ref, o1_ref):
    r = pl.program_id(1)
    ones_col = jnp.ones((_N, 1), jnp.float32)

    vt = jnp.swapaxes(adj_ref[0, :, 0, :, :], 0, 1).astype(jnp.float32)
    planes_ref[r] = vt

    @pl.when(r == 0)
    def _init():
        for b in range(_B):
            h = _dot_t(xt_ref[0, b], embw_ref[...]) + embb_ref[...]
            h_ref[b] = h
            o1_ref[b] = _mm(h, r1_ref[...]) + b1_ref[...]

    for b in range(_B):
        af = vt[b]
        inv = 1.0 / jnp.maximum(_dot_t(af, ones_col), 1.0)
        hw = _mm(h_ref[b], w1_ref[r])
        o1_ref[b] = o1_ref[b] + _dot_t(af, hw) * inv

    @pl.when(r == _R - 1)
    def _finish():
        for b in range(_B):
            h1 = jnp.maximum(o1_ref[b], 0.0)
            o = _mm(h1, r2_ref[...]) + b2_ref[...]
            for rr in range(_R):
                af = planes_ref[rr, b]
                inv = 1.0 / jnp.maximum(_dot_t(af, ones_col), 1.0)
                o = o + _dot_t(af, _mm(h1, w2_ref[rr])) * inv
            out_ref[0, b] = jnp.maximum(o, 0.0)


def kernel(unary_tensor, binary_tensor, emb_W, emb_b, W1, root1, b1, W2, root2, b2):
    xt = unary_tensor.astype(jnp.float32).transpose(0, 1, 3, 2)
    adj = binary_tensor.transpose(0, 2, 4, 1, 3)
    full = lambda *s: pl.BlockSpec(s, lambda t, r: (0,) * len(s))
    out = pl.pallas_call(
        _gnn_kernel,
        grid=(_T, _R),
        in_specs=[
            pl.BlockSpec((1, _B, _FEAT, _N), lambda t, r: (t, 0, 0, 0)),
            pl.BlockSpec((1, _N, 1, _B, _N), lambda t, r: (t, 0, r, 0, 0)),
            full(_FEAT, _EMB),
            full(1, _EMB),
            full(_R, _EMB, _EMB),
            full(_EMB, _EMB),
            full(1, _EMB),
            full(_R, _EMB, _EMB),
            full(_EMB, _EMB),
            full(1, _EMB),
        ],
        out_specs=pl.BlockSpec((1, _B, _N, _EMB), lambda t, r: (t, 0, 0, 0)),
        out_shape=jax.ShapeDtypeStruct((_T, _B, _N, _EMB), jnp.float32),
        scratch_shapes=[
            pltpu.VMEM((_R, _B, _N, _N), jnp.float32),
            pltpu.VMEM((_B, _N, _EMB), jnp.float32),
            pltpu.VMEM((_B, _N, _EMB), jnp.float32),
        ],
    )(xt, adj, emb_W, emb_b.reshape(1, _EMB), W1, root1, b1.reshape(1, _EMB),
      W2, root2, b2.reshape(1, _EMB))
    return out.reshape(_T * _B, _N * _EMB)


# speedup vs baseline: 1.2307x; 1.2307x over previous
import jax
import jax.numpy as jnp
from jax import lax
from jax.experimental import pallas as pl

_T, _B, _N, _FEAT, _R, _EMB = 2, 8, 300, 32, 3, 16


def _dot_t(a, b):
    return lax.dot_general(a, b, (((0,), (0,)), ((), ())),
                           preferred_element_type=jnp.float32)


def _gnn_kernel(xt_ref, adj_ref, embw_ref, embb_ref,
                w1_ref, r1_ref, b1_ref, w2_ref, r2_ref, b2_ref, out_ref):
    ones_col = jnp.ones((_N, 1), jnp.float32)
    mm = lambda a, c: jnp.dot(a, c, preferred_element_type=jnp.float32)

    hs = [_dot_t(xt_ref[0, b], embw_ref[...]) + embb_ref[...] for b in range(_B)]
    o1s = [mm(h, r1_ref[...]) + b1_ref[...] for h in hs]

    vts, invs = [], []
    for r in range(_R):
        vt = jnp.swapaxes(adj_ref[0, :, r, :, :], 0, 1).astype(jnp.float32)
        vts.append(vt)
        inv = [1.0 / jnp.maximum(_dot_t(vt[b], ones_col), 1.0) for b in range(_B)]
        invs.append(inv)
        for b in range(_B):
            o1s[b] = o1s[b] + _dot_t(vt[b], mm(hs[b], w1_ref[r])) * inv[b]

    for b in range(_B):
        h1 = jnp.maximum(o1s[b], 0.0)
        o = mm(h1, r2_ref[...]) + b2_ref[...]
        for r in range(_R):
            o = o + _dot_t(vts[r][b], mm(h1, w2_ref[r])) * invs[r][b]
        out_ref[0, b] = jnp.maximum(o, 0.0)


def kernel(unary_tensor, binary_tensor, emb_W, emb_b, W1, root1, b1, W2, root2, b2):
    xt = unary_tensor.astype(jnp.float32).transpose(0, 1, 3, 2)
    adj = binary_tensor.transpose(0, 2, 4, 1, 3)
    full = lambda *s: pl.BlockSpec(s, lambda t: (0,) * len(s))
    out = pl.pallas_call(
        _gnn_kernel,
        grid=(_T,),
        in_specs=[
            pl.BlockSpec((1, _B, _FEAT, _N), lambda t: (t, 0, 0, 0)),
            pl.BlockSpec((1, _N, _R, _B, _N), lambda t: (t, 0, 0, 0, 0)),
            full(_FEAT, _EMB),
            full(1, _EMB),
            full(_R, _EMB, _EMB),
            full(_EMB, _EMB),
            full(1, _EMB),
            full(_R, _EMB, _EMB),
            full(_EMB, _EMB),
            full(1, _EMB),
        ],
        out_specs=pl.BlockSpec((1, _B, _N, _EMB), lambda t: (t, 0, 0, 0)),
        out_shape=jax.ShapeDtypeStruct((_T, _B, _N, _EMB), jnp.float32),
    )(xt, adj, emb_W, emb_b.reshape(1, _EMB), W1, root1, b1.reshape(1, _EMB),
      W2, root2, b2.reshape(1, _EMB))
    return out.reshape(_T * _B, _N * _EMB)


# speedup vs baseline: 1.6974x; 1.3791x over previous
import jax
import jax.numpy as jnp
from jax import lax
from jax.experimental import pallas as pl

_T, _B, _N, _FEAT, _R, _EMB = 2, 8, 300, 32, 3, 16


def _dot_t(a, b):
    return lax.dot_general(a, b, (((0,), (0,)), ((), ())),
                           preferred_element_type=jnp.float32)


def _gnn_kernel(xt_ref, adj_ref, embw_ref, embb_ref,
                w1_ref, r1_ref, b1_ref, w2_ref, r2_ref, b2_ref, out_ref):
    mm = lambda a, c: jnp.dot(a, c, preferred_element_type=jnp.float32)

    hs = [_dot_t(embw_ref[...], xt_ref[0, b]) + embb_ref[...] for b in range(_B)]
    o1s = [_dot_t(r1_ref[...], h) + b1_ref[...] for h in hs]

    vts, invs = [], []
    for r in range(_R):
        vt = jnp.swapaxes(adj_ref[0, :, r, :, :], 0, 1).astype(jnp.float32)
        vts.append(vt)
        inv = [1.0 / jnp.maximum(jnp.sum(vt[b], axis=0, keepdims=True), 1.0)
               for b in range(_B)]
        invs.append(inv)
        for b in range(_B):
            o1s[b] = o1s[b] + mm(_dot_t(w1_ref[r], hs[b]), vts[r][b]) * inv[b]

    for b in range(_B):
        h1 = jnp.maximum(o1s[b], 0.0)
        o = _dot_t(r2_ref[...], h1) + b2_ref[...]
        for r in range(_R):
            o = o + mm(_dot_t(w2_ref[r], h1), vts[r][b]) * invs[r][b]
        out_ref[0, b] = jnp.maximum(o, 0.0)


def kernel(unary_tensor, binary_tensor, emb_W, emb_b, W1, root1, b1, W2, root2, b2):
    xt = unary_tensor.astype(jnp.float32).transpose(0, 1, 3, 2)
    adj = binary_tensor.transpose(0, 2, 4, 1, 3)
    full = lambda *s: pl.BlockSpec(s, lambda t: (0,) * len(s))
    out = pl.pallas_call(
        _gnn_kernel,
        grid=(_T,),
        in_specs=[
            pl.BlockSpec((1, _B, _FEAT, _N), lambda t: (t, 0, 0, 0)),
            pl.BlockSpec((1, _N, _R, _B, _N), lambda t: (t, 0, 0, 0, 0)),
            full(_FEAT, _EMB),
            full(_EMB, 1),
            full(_R, _EMB, _EMB),
            full(_EMB, _EMB),
            full(_EMB, 1),
            full(_R, _EMB, _EMB),
            full(_EMB, _EMB),
            full(_EMB, 1),
        ],
        out_specs=pl.BlockSpec((1, _B, _EMB, _N), lambda t: (t, 0, 0, 0)),
        out_shape=jax.ShapeDtypeStruct((_T, _B, _EMB, _N), jnp.float32),
    )(xt, adj, emb_W, emb_b.reshape(_EMB, 1), W1, root1, b1.reshape(_EMB, 1),
      W2, root2, b2.reshape(_EMB, 1))
    return out.transpose(0, 1, 3, 2).reshape(_T * _B, _N * _EMB)
